# baseline (speedup 1.0000x reference)
import functools

import jax
import jax.numpy as jnp
from jax.experimental import pallas as pl
from jax.experimental.pallas import tpu as pltpu


def _moe_body(x_ref, gw_ref, w1_ref, w3_ref, w2_ref, out_ref,
              h_ref, wbe_ref, xbf_ref, a1_ref, a3_ref, *, n_blk, blk, n_exp):
    t = pl.program_id(0)
    e = t // n_blk
    k = t % n_blk
    kb = k * blk
    B, E = wbe_ref.shape

    @pl.when(t == 0)
    def _gate():
        xv = x_ref[...]
        xbf_ref[...] = xv.astype(jnp.bfloat16)
        logits = jnp.dot(xv, gw_ref[...], preferred_element_type=jnp.float32)
        iota = jax.lax.broadcasted_iota(jnp.int32, (B, E), 1)
        i1 = jnp.argmax(logits, axis=1)
        first1 = iota == i1[:, None]
        m1 = jnp.max(logits, axis=1, keepdims=True)
        masked = jnp.where(first1, -jnp.inf, logits)
        i2 = jnp.argmax(masked, axis=1)
        first2 = iota == i2[:, None]
        m2 = jnp.max(masked, axis=1, keepdims=True)
        d = jnp.exp(m2 - m1)
        s1 = 1.0 / (1.0 + d)
        s2 = 1.0 - s1
        wbe_ref[...] = jnp.where(first1, s1, 0.0) + jnp.where(first2, s2, 0.0)

    @pl.when(e < n_exp)
    def _hidden():
        xb = xbf_ref[:, pl.ds(kb, blk)]
        p1 = jnp.dot(xb, w1_ref[0].astype(jnp.bfloat16),
                     preferred_element_type=jnp.float32)
        p3 = jnp.dot(xb, w3_ref[0].astype(jnp.bfloat16),
                     preferred_element_type=jnp.float32)

        @pl.when(k == 0)
        def _init():
            a1_ref[...] = p1
            a3_ref[...] = p3

        @pl.when(k > 0)
        def _acc():
            a1_ref[...] += p1
            a3_ref[...] += p3

        @pl.when(k == n_blk - 1)
        def _finish():
            a1 = a1_ref[...]
            a3 = a3_ref[...]
            iota = jax.lax.broadcasted_iota(jnp.int32, (B, E), 1)
            wcol = jnp.sum(jnp.where(iota == e, wbe_ref[...], 0.0),
                           axis=1, keepdims=True)
            hv = a1 * jax.nn.sigmoid(a1) * a3 * wcol
            h_ref[pl.ds(e % 2, 1), :, :] = hv.astype(jnp.bfloat16)[None]

    @pl.when(e >= 1)
    def _combine():
        hp = h_ref[pl.ds((e - 1) % 2, 1), :, pl.ds(kb, blk)][0]
        contrib = jnp.dot(hp, w2_ref[0].astype(jnp.bfloat16),
                          preferred_element_type=jnp.float32)

        @pl.when((e == 1) & (k == 0))
        def _init():
            out_ref[...] = contrib

        @pl.when((e > 1) | (k > 0))
        def _acc():
            out_ref[...] += contrib


def kernel(x, gate_w, w1, w3, w2):
    B, H = x.shape[2], x.shape[3]
    E = gate_w.shape[1]
    F = w1.shape[2]
    BLK = 512
    N = H // BLK
    xt = x.reshape(B, H)

    def w13_map(t):
        e, k = t // N, t % N
        ep = e == E
        return (jnp.where(ep, E - 1, e), jnp.where(ep, N - 1, k), 0)

    def w2_map(t):
        e, k = t // N, t % N
        return (jnp.maximum(e, 1) - 1, jnp.where(e == 0, 0, k), 0)

    out = pl.pallas_call(
        functools.partial(_moe_body, n_blk=N, blk=BLK, n_exp=E),
        grid=((E + 1) * N,),
        in_specs=[
            pl.BlockSpec((B, H), lambda t: (0, 0)),
            pl.BlockSpec((H, E), lambda t: (0, 0)),
            pl.BlockSpec((1, BLK, F), w13_map),
            pl.BlockSpec((1, BLK, F), w13_map),
            pl.BlockSpec((1, BLK, H), w2_map),
        ],
        out_specs=pl.BlockSpec((B, H), lambda t: (0, 0)),
        out_shape=jax.ShapeDtypeStruct((B, H), jnp.float32),
        scratch_shapes=[
            pltpu.VMEM((2, B, F), jnp.bfloat16),
            pltpu.VMEM((B, E), jnp.float32),
            pltpu.VMEM((B, H), jnp.bfloat16),
            pltpu.VMEM((B, F), jnp.float32),
            pltpu.VMEM((B, F), jnp.float32),
        ],
        compiler_params=pltpu.CompilerParams(
            dimension_semantics=("arbitrary",),
        ),
    )(xt, gate_w, w1, w3, w2)
    return out.reshape(1, 1, B, H)


# speedup vs baseline: 1.0044x; 1.0044x over previous
import functools

import jax
import jax.numpy as jnp
from jax.experimental import pallas as pl
from jax.experimental.pallas import tpu as pltpu


def _moe_body(x_ref, gw_ref, w1_ref, w3_ref, w2_ref, out_ref,
              h_ref, wbe_ref, xbf_ref, a1_ref, a3_ref, *, n_blk, blk, n_exp):
    t = pl.program_id(0)
    e = t // n_blk
    k = t % n_blk
    kb = k * blk
    B, E = wbe_ref.shape

    @pl.when(t == 0)
    def _gate():
        xv = x_ref[...]
        xbf_ref[...] = xv.astype(jnp.bfloat16)
        logits = jnp.dot(xv, gw_ref[...], preferred_element_type=jnp.float32)
        iota = jax.lax.broadcasted_iota(jnp.int32, (B, E), 1)
        i1 = jnp.argmax(logits, axis=1)
        first1 = iota == i1[:, None]
        m1 = jnp.max(logits, axis=1, keepdims=True)
        masked = jnp.where(first1, -jnp.inf, logits)
        i2 = jnp.argmax(masked, axis=1)
        first2 = iota == i2[:, None]
        m2 = jnp.max(masked, axis=1, keepdims=True)
        d = jnp.exp(m2 - m1)
        s1 = 1.0 / (1.0 + d)
        s2 = 1.0 - s1
        wbe_ref[...] = jnp.where(first1, s1, 0.0) + jnp.where(first2, s2, 0.0)

    @pl.when(e < n_exp)
    def _hidden():
        xb = xbf_ref[:, pl.ds(kb, blk)]
        p1 = jnp.dot(xb, w1_ref[0].astype(jnp.bfloat16),
                     preferred_element_type=jnp.float32)
        p3 = jnp.dot(xb, w3_ref[0].astype(jnp.bfloat16),
                     preferred_element_type=jnp.float32)

        @pl.when(k == 0)
        def _init():
            a1_ref[...] = p1
            a3_ref[...] = p3

        @pl.when(k > 0)
        def _acc():
            a1_ref[...] += p1
            a3_ref[...] += p3

        @pl.when(k == n_blk - 1)
        def _finish():
            a1 = a1_ref[...]
            a3 = a3_ref[...]
            iota = jax.lax.broadcasted_iota(jnp.int32, (B, E), 1)
            wcol = jnp.sum(jnp.where(iota == e, wbe_ref[...], 0.0),
                           axis=1, keepdims=True)
            hv = a1 * jax.nn.sigmoid(a1) * a3 * wcol
            h_ref[pl.ds(e % 2, 1), :, :] = hv.astype(jnp.bfloat16)[None]

    @pl.when(e >= 1)
    def _combine():
        hp = h_ref[pl.ds((e - 1) % 2, 1), :, pl.ds(kb, blk)][0]
        contrib = jnp.dot(hp, w2_ref[0].astype(jnp.bfloat16),
                          preferred_element_type=jnp.float32)

        @pl.when((e == 1) & (k == 0))
        def _init():
            out_ref[...] = contrib

        @pl.when((e > 1) | (k > 0))
        def _acc():
            out_ref[...] += contrib


def kernel(x, gate_w, w1, w3, w2):
    B, H = x.shape[2], x.shape[3]
    E = gate_w.shape[1]
    F = w1.shape[2]
    BLK = 256
    N = H // BLK
    xt = x.reshape(B, H)

    def w13_map(t):
        e, k = t // N, t % N
        ep = e == E
        return (jnp.where(ep, E - 1, e), jnp.where(ep, N - 1, k), 0)

    def w2_map(t):
        e, k = t // N, t % N
        return (jnp.maximum(e, 1) - 1, jnp.where(e == 0, 0, k), 0)

    out = pl.pallas_call(
        functools.partial(_moe_body, n_blk=N, blk=BLK, n_exp=E),
        grid=((E + 1) * N,),
        in_specs=[
            pl.BlockSpec((B, H), lambda t: (0, 0)),
            pl.BlockSpec((H, E), lambda t: (0, 0)),
            pl.BlockSpec((1, BLK, F), w13_map),
            pl.BlockSpec((1, BLK, F), w13_map),
            pl.BlockSpec((1, BLK, H), w2_map),
        ],
        out_specs=pl.BlockSpec((B, H), lambda t: (0, 0)),
        out_shape=jax.ShapeDtypeStruct((B, H), jnp.float32),
        scratch_shapes=[
            pltpu.VMEM((2, B, F), jnp.bfloat16),
            pltpu.VMEM((B, E), jnp.float32),
            pltpu.VMEM((B, H), jnp.bfloat16),
            pltpu.VMEM((B, F), jnp.float32),
            pltpu.VMEM((B, F), jnp.float32),
        ],
        compiler_params=pltpu.CompilerParams(
            dimension_semantics=("arbitrary",),
        ),
    )(xt, gate_w, w1, w3, w2)
    return out.reshape(1, 1, B, H)
